# Initial kernel scaffold; baseline (speedup 1.0000x reference)
import jax
import jax.numpy as jnp
from jax.experimental import pallas as pl


def kernel(x, W):
    raise NotImplementedError("write your pallas kernel here")


# speedup vs baseline: 8.7177x; 8.7177x over previous
import functools

import jax
import jax.numpy as jnp
from jax import lax
from jax.experimental import pallas as pl
from jax.experimental.pallas import tpu as pltpu
from jax.experimental.pallas import tpu_sc as plsc

VOCAB = 1000000
D = 32
B = 16384
H = 200

NC = 2
NS = 16
NW = NC * NS
RPW = B // NW
IC = 32
G = 40
SPR = H // G
NGRP = RPW // IC

_mesh = plsc.VectorSubcoreMesh(
    core_axis_name="c", subcore_axis_name="s", num_cores=NC, num_subcores=NS
)


@functools.partial(
    pl.kernel,
    out_type=jax.ShapeDtypeStruct((B, D), jnp.float32),
    mesh=_mesh,
    compiler_params=pltpu.CompilerParams(use_tc_tiling_on_sc=False),
    scratch_types=[
        pltpu.VMEM((IC * H,), jnp.int32),
        pltpu.VMEM((H, D), jnp.float32),
        pltpu.VMEM((IC, D), jnp.float32),
        pltpu.SemaphoreType.DMA,
    ],
)
def _encode(x_hbm, w_hbm, out_hbm, idx_v, rows_v, out_v, sem):
    wid = lax.axis_index("s") * NC + lax.axis_index("c")
    base_row = wid * RPW
    inv_h = jnp.float32(1.0 / H)

    def group_body(g, _):
        grp_row = base_row + g * IC
        pltpu.sync_copy(x_hbm.at[pl.ds(grp_row * H, IC * H)], idx_v)

        def row_body(r, _):
            cps = [
                pltpu.async_copy(
                    w_hbm.at[idx_v.at[pl.ds(r * H + i * G, G)]],
                    rows_v.at[pl.ds(i * G, G)],
                    sem,
                )
                for i in range(SPR)
            ]
            for cp in cps:
                cp.wait()

            def acc_body(j, carry):
                a0, a1 = carry
                a0 = a0 + rows_v[j, pl.ds(0, 16)]
                a1 = a1 + rows_v[j, pl.ds(16, 16)]
                return a0, a1

            a0, a1 = lax.fori_loop(
                0, H, acc_body,
                (jnp.zeros((16,), jnp.float32), jnp.zeros((16,), jnp.float32)),
            )
            out_v[r, pl.ds(0, 16)] = a0 * inv_h
            out_v[r, pl.ds(16, 16)] = a1 * inv_h
            return 0

        lax.fori_loop(0, IC, row_body, 0)
        pltpu.sync_copy(out_v, out_hbm.at[pl.ds(grp_row, IC)])
        return 0

    lax.fori_loop(0, NGRP, group_body, 0)


def kernel(x, W):
    return _encode(x.reshape(-1), W)


# speedup vs baseline: 14.6446x; 1.6799x over previous
import functools

import jax
import jax.numpy as jnp
from jax import lax
from jax.experimental import pallas as pl
from jax.experimental.pallas import tpu as pltpu
from jax.experimental.pallas import tpu_sc as plsc

VOCAB = 1000000
D = 32
B = 16384
H = 200

NC = 2
NS = 16
NW = NC * NS
RPW = B // NW
IC = 32
NGRP = RPW // IC
G0 = 128
G1 = H - G0
U = 8
NACC = 4

_mesh = plsc.VectorSubcoreMesh(
    core_axis_name="c", subcore_axis_name="s", num_cores=NC, num_subcores=NS
)


@functools.partial(
    pl.kernel,
    out_type=jax.ShapeDtypeStruct((B, D), jnp.float32),
    mesh=_mesh,
    compiler_params=pltpu.CompilerParams(use_tc_tiling_on_sc=False),
    scratch_types=[
        pltpu.VMEM((IC * H,), jnp.int32),
        pltpu.VMEM((2, H, D), jnp.float32),
        pltpu.VMEM((IC, D), jnp.float32),
        pltpu.SemaphoreType.DMA,
        pltpu.SemaphoreType.DMA,
    ],
)
def _encode(x_hbm, w_hbm, out_hbm, idx_v, rows_v, out_v, sem0, sem1):
    wid = lax.axis_index("s") * NC + lax.axis_index("c")
    base_row = wid * RPW
    inv_h = jnp.float32(1.0 / H)
    sems = (sem0, sem1)

    def fire(r, b):
        cp0 = pltpu.async_copy(
            w_hbm.at[idx_v.at[pl.ds(r * H, G0)]],
            rows_v.at[b].at[pl.ds(0, G0)],
            sems[b],
        )
        cp1 = pltpu.async_copy(
            w_hbm.at[idx_v.at[pl.ds(r * H + G0, G1)]],
            rows_v.at[b].at[pl.ds(G0, G1)],
            sems[b],
        )
        return cp0, cp1

    def drain(r, b):
        cp0 = pltpu.make_async_copy(
            w_hbm.at[idx_v.at[pl.ds(r * H, G0)]],
            rows_v.at[b].at[pl.ds(0, G0)],
            sems[b],
        )
        cp1 = pltpu.make_async_copy(
            w_hbm.at[idx_v.at[pl.ds(r * H + G0, G1)]],
            rows_v.at[b].at[pl.ds(G0, G1)],
            sems[b],
        )
        cp0.wait()
        cp1.wait()

    def accumulate(b):
        def acc_body(it, carry):
            a = list(carry)
            base = it * U
            for u in range(U):
                j = base + u
                k = u % NACC
                a[2 * k] = a[2 * k] + rows_v[b, j, pl.ds(0, 16)]
                a[2 * k + 1] = a[2 * k + 1] + rows_v[b, j, pl.ds(16, 16)]
            return tuple(a)

        zeros = tuple(jnp.zeros((16,), jnp.float32) for _ in range(2 * NACC))
        a = lax.fori_loop(0, H // U, acc_body, zeros)
        lo = (a[0] + a[2]) + (a[4] + a[6])
        hi = (a[1] + a[3]) + (a[5] + a[7])
        return lo * inv_h, hi * inv_h

    def group_body(g, _):
        grp_row = base_row + g * IC
        pltpu.sync_copy(x_hbm.at[pl.ds(grp_row * H, IC * H)], idx_v)
        fire(0, 0)
        fire(1, 1)

        def pair_body(rr, _):
            for b in range(2):
                r = rr * 2 + b
                drain(r, b)

                @pl.when(rr < IC // 2 - 1)
                def _():
                    fire(r + 2, b)

                lo, hi = accumulate(b)
                out_v[r, pl.ds(0, 16)] = lo
                out_v[r, pl.ds(16, 16)] = hi
            return 0

        lax.fori_loop(0, IC // 2, pair_body, 0)
        pltpu.sync_copy(out_v, out_hbm.at[pl.ds(grp_row, IC)])
        return 0

    lax.fori_loop(0, NGRP, group_body, 0)


def kernel(x, W):
    return _encode(x.reshape(-1), W)


# speedup vs baseline: 16.4251x; 1.1216x over previous
import functools

import jax
import jax.numpy as jnp
from jax import lax
from jax.experimental import pallas as pl
from jax.experimental.pallas import tpu as pltpu
from jax.experimental.pallas import tpu_sc as plsc

VOCAB = 1000000
D = 32
B = 16384
H = 200

NC = 2
NS = 16
NW = NC * NS
RPW = B // NW
IC = 32
NGRP = RPW // IC
G0 = 128
G1 = H - G0
U = 8
NACC = 4
NBUF = 4

_mesh = plsc.VectorSubcoreMesh(
    core_axis_name="c", subcore_axis_name="s", num_cores=NC, num_subcores=NS
)


@functools.partial(
    pl.kernel,
    out_type=jax.ShapeDtypeStruct((B, D), jnp.float32),
    mesh=_mesh,
    compiler_params=pltpu.CompilerParams(use_tc_tiling_on_sc=False),
    scratch_types=[
        pltpu.VMEM((IC, H), jnp.int32),
        pltpu.VMEM((NBUF, H, D), jnp.float32),
        pltpu.VMEM((IC, D), jnp.float32),
    ]
    + [pltpu.SemaphoreType.DMA for _ in range(NBUF)],
)
def _encode(x_hbm, w_hbm, out_hbm, idx_v, rows_v, out_v, *sems):
    wid = lax.axis_index("s") * NC + lax.axis_index("c")
    base_row = wid * RPW
    inv_h = jnp.float32(1.0 / H)

    def copies(r, b):
        row_idx = idx_v.at[r]
        cp0 = pltpu.make_async_copy(
            w_hbm.at[row_idx.at[pl.ds(0, G0)]],
            rows_v.at[b].at[pl.ds(0, G0)],
            sems[b],
        )
        cp1 = pltpu.make_async_copy(
            w_hbm.at[row_idx.at[pl.ds(G0, G1)]],
            rows_v.at[b].at[pl.ds(G0, G1)],
            sems[b],
        )
        return cp0, cp1

    def fire(r, b):
        for cp in copies(r, b):
            cp.start()

    def drain(r, b):
        for cp in copies(r, b):
            cp.wait()

    def accumulate(b):
        def acc_body(it, carry):
            a = list(carry)
            base = it * U
            for u in range(U):
                j = base + u
                k = u % NACC
                a[2 * k] = a[2 * k] + rows_v[b, j, pl.ds(0, 16)]
                a[2 * k + 1] = a[2 * k + 1] + rows_v[b, j, pl.ds(16, 16)]
            return tuple(a)

        zeros = tuple(jnp.zeros((16,), jnp.float32) for _ in range(2 * NACC))
        a = lax.fori_loop(0, H // U, acc_body, zeros)
        lo = (a[0] + a[2]) + (a[4] + a[6])
        hi = (a[1] + a[3]) + (a[5] + a[7])
        return lo * inv_h, hi * inv_h

    def group_body(g, _):
        grp_row = base_row + g * IC
        pltpu.sync_copy(x_hbm.at[pl.ds(grp_row, IC)], idx_v)
        for b in range(NBUF):
            fire(b, b)

        def ring_body(rr, _):
            for b in range(NBUF):
                r = rr * NBUF + b
                drain(r, b)

                @pl.when(rr < IC // NBUF - 1)
                def _():
                    fire(r + NBUF, b)

                lo, hi = accumulate(b)
                out_v[r, pl.ds(0, 16)] = lo
                out_v[r, pl.ds(16, 16)] = hi
            return 0

        lax.fori_loop(0, IC // NBUF, ring_body, 0)
        pltpu.sync_copy(out_v, out_hbm.at[pl.ds(grp_row, IC)])
        return 0

    lax.fori_loop(0, NGRP, group_body, 0)


def kernel(x, W):
    return _encode(x, W)
